# baseline (speedup 1.0000x reference)
import functools

import jax
import jax.numpy as jnp
from jax import lax
from jax.experimental import pallas as pl
from jax.experimental.pallas import tpu as pltpu
from jax.experimental.pallas import tpu_sc as plsc

D = 64
NB = 4096
NS_ = 50
V = 100000

_info = plsc.get_sparse_core_info()
_NC = _info.num_cores
_NSUB = _info.num_subcores
NW = _NC * _NSUB
NPH = D // NW
NGRP = NB // 16

_mesh = plsc.VectorSubcoreMesh(core_axis_name="c", subcore_axis_name="s")


@functools.partial(
    pl.kernel,
    mesh=_mesh,
    out_type=jax.ShapeDtypeStruct((NS_, D, NB), jnp.float32),
    scratch_types=[
        pltpu.VMEM((V,), jnp.float32),
        pltpu.VMEM((2, NB), jnp.int32),
        pltpu.VMEM((2, NB), jnp.float32),
        pltpu.VMEM_SHARED((NS_ * NB,), jnp.int32),
        pltpu.SemaphoreType.DMA,
        pltpu.SemaphoreType.DMA((2,)),
        pltpu.SemaphoreType.DMA((2,)),
    ],
    compiler_params=pltpu.CompilerParams(needs_layout_passes=False),
)
def _lookup_kernel(cat_hbm, tab_hbm, out_hbm, row_v, idx_v, res_v, cat_s,
                   rsem, isem, osem):
    sid = lax.axis_index("s")
    wid = sid * _NC + lax.axis_index("c")

    for j in range(4):
        s = sid + 16 * j

        @pl.when(s < NS_)
        def _():
            pltpu.sync_copy(cat_hbm.at[s], idx_v.at[j % 2])
            pltpu.sync_copy(idx_v.at[j % 2], cat_s.at[pl.ds(s * NB, NB)])

    plsc.subcore_barrier()

    for p in range(NPH):
        d = wid + p * NW
        pltpu.async_copy(tab_hbm.at[d], row_v, rsem)
        for b in range(2):
            pltpu.async_copy(cat_s.at[pl.ds(b * NB, NB)], idx_v.at[b], isem.at[b])
        pltpu.make_async_copy(tab_hbm.at[d], row_v, rsem).wait()

        def body(k, carry):
            for b in range(2):
                s = 2 * k + b
                soff = pl.multiple_of(s * NB, NB)
                pltpu.make_async_copy(
                    cat_s.at[pl.ds(soff, NB)], idx_v.at[b], isem.at[b]).wait()

                @pl.when(k > 0)
                def _():
                    pltpu.make_async_copy(
                        res_v.at[b], out_hbm.at[s, d], osem.at[b]).wait()

                @plsc.parallel_loop(0, NGRP, unroll=32)
                def grp(g):
                    off = pl.multiple_of(g * 16, 16)
                    idx = idx_v[b, pl.ds(off, 16)]
                    res_v[b, pl.ds(off, 16)] = plsc.load_gather(row_v, [idx])
                pltpu.async_copy(res_v.at[b], out_hbm.at[s, d], osem.at[b])

                @pl.when(s + 2 < NS_)
                def _():
                    pltpu.async_copy(
                        cat_s.at[pl.ds(pl.multiple_of((s + 2) * NB, NB), NB)],
                        idx_v.at[b], isem.at[b])
            return carry

        lax.fori_loop(0, NS_ // 2, body, 0)
        for b in range(2):
            pltpu.make_async_copy(
                res_v.at[b], out_hbm.at[0, d], osem.at[b]).wait()


def kernel(category, weight):
    out = _lookup_kernel(category.T, weight.T)
    return out.transpose(2, 0, 1)


# speedup vs baseline: 1.0226x; 1.0226x over previous
import functools

import jax
import jax.numpy as jnp
from jax import lax
from jax.experimental import pallas as pl
from jax.experimental.pallas import tpu as pltpu
from jax.experimental.pallas import tpu_sc as plsc

D = 64
NB = 4096
NS_ = 50
V = 100000

_info = plsc.get_sparse_core_info()
_NC = _info.num_cores
_NSUB = _info.num_subcores
NW = _NC * _NSUB
NPH = D // NW
NGRP = NB // 16

_mesh = plsc.VectorSubcoreMesh(core_axis_name="c", subcore_axis_name="s")


@functools.partial(
    pl.kernel,
    mesh=_mesh,
    out_type=jax.ShapeDtypeStruct((NS_, D, NB), jnp.float32),
    scratch_types=[
        pltpu.VMEM((V,), jnp.float32),
        pltpu.VMEM((2, NB), jnp.int32),
        pltpu.VMEM((2, NB), jnp.float32),
        pltpu.VMEM_SHARED((NS_ * NB,), jnp.int32),
        pltpu.SemaphoreType.DMA,
        pltpu.SemaphoreType.DMA((2,)),
        pltpu.SemaphoreType.DMA((2,)),
    ],
    compiler_params=pltpu.CompilerParams(needs_layout_passes=False),
)
def _lookup_kernel(cat_hbm, tab_hbm, out_hbm, row_v, idx_v, res_v, cat_s,
                   rsem, isem, osem):
    sid = lax.axis_index("s")
    wid = sid * _NC + lax.axis_index("c")
    pltpu.async_copy(tab_hbm.at[wid], row_v, rsem)

    for j in range(4):
        s = sid + 16 * j

        @pl.when(s < NS_)
        def _():
            pltpu.sync_copy(cat_hbm.at[s], idx_v.at[j % 2])
            pltpu.sync_copy(idx_v.at[j % 2], cat_s.at[pl.ds(s * NB, NB)])

    plsc.subcore_barrier()

    for p in range(NPH):
        d = wid + p * NW
        if p > 0:
            pltpu.async_copy(tab_hbm.at[d], row_v, rsem)
        for b in range(2):
            pltpu.async_copy(cat_s.at[pl.ds(b * NB, NB)], idx_v.at[b], isem.at[b])
        pltpu.make_async_copy(tab_hbm.at[d], row_v, rsem).wait()

        def body(k, carry):
            for b in range(2):
                s = 2 * k + b
                soff = pl.multiple_of(s * NB, NB)
                pltpu.make_async_copy(
                    cat_s.at[pl.ds(soff, NB)], idx_v.at[b], isem.at[b]).wait()

                @pl.when(k > 0)
                def _():
                    pltpu.make_async_copy(
                        res_v.at[b], out_hbm.at[s, d], osem.at[b]).wait()

                @plsc.parallel_loop(0, NGRP, unroll=32)
                def grp(g):
                    off = pl.multiple_of(g * 16, 16)
                    idx = idx_v[b, pl.ds(off, 16)]
                    res_v[b, pl.ds(off, 16)] = plsc.load_gather(row_v, [idx])
                pltpu.async_copy(res_v.at[b], out_hbm.at[s, d], osem.at[b])

                @pl.when(s + 2 < NS_)
                def _():
                    pltpu.async_copy(
                        cat_s.at[pl.ds(pl.multiple_of((s + 2) * NB, NB), NB)],
                        idx_v.at[b], isem.at[b])
            return carry

        lax.fori_loop(0, NS_ // 2, body, 0)
        for b in range(2):
            pltpu.make_async_copy(
                res_v.at[b], out_hbm.at[0, d], osem.at[b]).wait()


def kernel(category, weight):
    out = _lookup_kernel(category.T, weight.T)
    return out.transpose(2, 0, 1)
